# baseline (speedup 1.0000x reference)
import functools

import jax
import jax.numpy as jnp
from jax import lax
from jax.experimental import pallas as pl
from jax.experimental.pallas import tpu as pltpu
from jax.experimental.pallas import tpu_sc as plsc


_NROWS = 4096
_NCOLS = 8192
_NCORES = 2
_NSUB = 16
_NW = _NCORES * _NSUB
_RPW = _NROWS // _NW
_NV = _NCOLS // 16


@functools.partial(
    pl.kernel,
    out_type=jax.ShapeDtypeStruct((_NROWS, _NCOLS), jnp.float32),
    mesh=plsc.VectorSubcoreMesh(core_axis_name="c", subcore_axis_name="s"),
    scratch_types=[pltpu.VMEM((_NCOLS,), jnp.float32)],
    compiler_params=pltpu.CompilerParams(needs_layout_passes=False),
)
def _sc_cumsum(x_hbm, o_hbm, buf):
    wid = lax.axis_index("s") * _NCORES + lax.axis_index("c")
    base = wid * _RPW

    def row_body(r, _):
        row = base + r
        pltpu.sync_copy(x_hbm.at[row], buf)

        def vec_body(i, carry):
            v = buf[pl.ds(i * 16, 16)]
            s = plsc.cumsum(v)
            buf[pl.ds(i * 16, 16)] = s + carry
            return carry + jnp.sum(v)

        lax.fori_loop(0, _NV, vec_body, jnp.float32(0.0))
        pltpu.sync_copy(buf, o_hbm.at[row])
        return _

    lax.fori_loop(0, _RPW, row_body, 0)


_BR = 2048
_BC = 1024
_CH = 128


def _tc_scan_kernel(x_ref, tri_ref, o_ref, carry_ref):
    j = pl.program_id(1)

    @pl.when(j == 0)
    def _():
        carry_ref[...] = jnp.zeros_like(carry_ref)

    tri = tri_ref[...]
    carry = carry_ref[:, 0:1]
    for k in range(_BC // _CH):
        xc = x_ref[:, k * _CH : (k + 1) * _CH]
        part = jax.lax.dot_general(
            xc,
            tri,
            dimension_numbers=(((1,), (0,)), ((), ())),
            precision=jax.lax.Precision.DEFAULT,
            preferred_element_type=jnp.float32,
        )
        outc = part + carry
        o_ref[:, k * _CH : (k + 1) * _CH] = outc
        carry = outc[:, _CH - 1 : _CH]
    carry_ref[...] = jnp.broadcast_to(carry, carry_ref.shape)


def _tc_cumsum(x):
    m, n = x.shape
    tri = jnp.triu(jnp.ones((_CH, _CH), dtype=jnp.float32))
    return pl.pallas_call(
        _tc_scan_kernel,
        grid=(m // _BR, n // _BC),
        in_specs=[
            pl.BlockSpec((_BR, _BC), lambda i, j: (i, j)),
            pl.BlockSpec((_CH, _CH), lambda i, j: (0, 0)),
        ],
        out_specs=pl.BlockSpec((_BR, _BC), lambda i, j: (i, j)),
        out_shape=jax.ShapeDtypeStruct((m, n), jnp.float32),
        scratch_shapes=[pltpu.VMEM((_BR, 128), jnp.float32)],
        compiler_params=pltpu.CompilerParams(
            dimension_semantics=("parallel", "arbitrary"),
        ),
    )(x, tri)


def kernel(x):
    return _sc_cumsum(x)


# speedup vs baseline: 3.4350x; 3.4350x over previous
import functools

import jax
import jax.numpy as jnp
from jax import lax
from jax.experimental import pallas as pl
from jax.experimental.pallas import tpu as pltpu
from jax.experimental.pallas import tpu_sc as plsc


_NROWS = 4096
_NCOLS = 8192
_NCORES = 2
_NSUB = 16
_NW = _NCORES * _NSUB
_RPW = _NROWS // _NW
_NV = _NCOLS // 16


_G = 8
_HC = _NCOLS // 2
_NTASK = (_RPW // _G) * 2


@functools.partial(
    pl.kernel,
    out_type=jax.ShapeDtypeStruct((_NROWS, _NCOLS), jnp.float32),
    mesh=plsc.VectorSubcoreMesh(core_axis_name="c", subcore_axis_name="s"),
    scratch_types=[
        pltpu.VMEM((_G, _HC), jnp.float32),
        pltpu.VMEM((_G, _HC), jnp.float32),
        pltpu.SemaphoreType.DMA,
        pltpu.SemaphoreType.DMA,
        pltpu.SemaphoreType.DMA,
        pltpu.SemaphoreType.DMA,
    ],
    compiler_params=pltpu.CompilerParams(needs_layout_passes=False),
)
def _sc_cumsum(x_hbm, o_hbm, buf0, buf1, in0, in1, out0, out1):
    wid = lax.axis_index("s") * _NCORES + lax.axis_index("c")
    base = wid * _RPW
    bufs = (buf0, buf1)
    in_sems = (in0, in1)
    out_sems = (out0, out1)
    last15 = jnp.full((16, 1), 15, dtype=jnp.int32)
    bcast_dnums = lax.GatherDimensionNumbers(
        offset_dims=(), collapsed_slice_dims=(0,), start_index_map=(0,)
    )

    def bcast_last(v):
        return lax.gather(
            v,
            last15,
            dimension_numbers=bcast_dnums,
            slice_sizes=(1,),
            mode=lax.GatherScatterMode.PROMISE_IN_BOUNDS,
        )

    def src(t):
        g, h = t // 2, t % 2
        return x_hbm.at[pl.ds(base + g * _G, _G), pl.ds(h * _HC, _HC)]

    def dst(t):
        g, h = t // 2, t % 2
        return o_hbm.at[pl.ds(base + g * _G, _G), pl.ds(h * _HC, _HC)]

    in_handles = [None] * _NTASK
    out_handles = [None] * _NTASK
    in_handles[0] = pltpu.async_copy(src(0), bufs[0], in_sems[0])

    carrys = None
    for t in range(_NTASK):
        b = t % 2
        buf = bufs[b]
        if t + 1 < _NTASK:
            if t >= 1:
                out_handles[t - 1].wait()
            in_handles[t + 1] = pltpu.async_copy(
                src(t + 1), bufs[(t + 1) % 2], in_sems[(t + 1) % 2]
            )
        in_handles[t].wait()

        if t % 2 == 0:
            carrys = tuple(jnp.zeros((16,), jnp.float32) for _ in range(_G))

        def vec_body(i, cs):
            new = []
            for r in range(_G):
                v = buf[r, pl.ds(i * 16, 16)]
                out = plsc.cumsum(v) + cs[r]
                buf[r, pl.ds(i * 16, 16)] = out
                new.append(bcast_last(out))
            return tuple(new)

        carrys = lax.fori_loop(0, _HC // 16, vec_body, carrys)
        out_handles[t] = pltpu.async_copy(buf, dst(t), out_sems[b])

    out_handles[_NTASK - 2].wait()
    out_handles[_NTASK - 1].wait()


_BR = 2048
_BC = 1024
_CH = 128


def _tc_scan_kernel(x_ref, tri_ref, o_ref, carry_ref):
    j = pl.program_id(1)

    @pl.when(j == 0)
    def _():
        carry_ref[...] = jnp.zeros_like(carry_ref)

    tri = tri_ref[...]
    carry = carry_ref[:, 0:1]
    for k in range(_BC // _CH):
        xc = x_ref[:, k * _CH : (k + 1) * _CH]
        part = jax.lax.dot_general(
            xc,
            tri,
            dimension_numbers=(((1,), (0,)), ((), ())),
            precision=jax.lax.Precision.DEFAULT,
            preferred_element_type=jnp.float32,
        )
        outc = part + carry
        o_ref[:, k * _CH : (k + 1) * _CH] = outc
        carry = outc[:, _CH - 1 : _CH]
    carry_ref[...] = jnp.broadcast_to(carry, carry_ref.shape)


def _tc_cumsum(x):
    m, n = x.shape
    tri = jnp.triu(jnp.ones((_CH, _CH), dtype=jnp.float32))
    return pl.pallas_call(
        _tc_scan_kernel,
        grid=(m // _BR, n // _BC),
        in_specs=[
            pl.BlockSpec((_BR, _BC), lambda i, j: (i, j)),
            pl.BlockSpec((_CH, _CH), lambda i, j: (0, 0)),
        ],
        out_specs=pl.BlockSpec((_BR, _BC), lambda i, j: (i, j)),
        out_shape=jax.ShapeDtypeStruct((m, n), jnp.float32),
        scratch_shapes=[pltpu.VMEM((_BR, 128), jnp.float32)],
        compiler_params=pltpu.CompilerParams(
            dimension_semantics=("parallel", "arbitrary"),
        ),
    )(x, tri)


def kernel(x):
    return _sc_cumsum(x)


# speedup vs baseline: 5.1132x; 1.4886x over previous
import functools

import jax
import jax.numpy as jnp
from jax import lax
from jax.experimental import pallas as pl
from jax.experimental.pallas import tpu as pltpu
from jax.experimental.pallas import tpu_sc as plsc


_NROWS = 4096
_NCOLS = 8192
_NCORES = 2
_NSUB = 16
_NW = _NCORES * _NSUB
_RPW = _NROWS // _NW
_NV = _NCOLS // 16


_G = 8
_HC = _NCOLS // 2
_NTASK = (_RPW // _G) * 2


@functools.partial(
    pl.kernel,
    out_type=jax.ShapeDtypeStruct((_NROWS, _NCOLS), jnp.float32),
    mesh=plsc.VectorSubcoreMesh(core_axis_name="c", subcore_axis_name="s"),
    scratch_types=[
        pltpu.VMEM((_G, _HC), jnp.float32),
        pltpu.VMEM((_G, _HC), jnp.float32),
        pltpu.SemaphoreType.DMA,
        pltpu.SemaphoreType.DMA,
        pltpu.SemaphoreType.DMA,
        pltpu.SemaphoreType.DMA,
    ],
    compiler_params=pltpu.CompilerParams(needs_layout_passes=False),
)
def _sc_cumsum(x_hbm, o_hbm, buf0, buf1, in0, in1, out0, out1):
    wid = lax.axis_index("s") * _NCORES + lax.axis_index("c")
    base = wid * _RPW
    bufs = (buf0, buf1)
    in_sems = (in0, in1)
    out_sems = (out0, out1)
    last15 = jnp.full((16, 1), 15, dtype=jnp.int32)
    bcast_dnums = lax.GatherDimensionNumbers(
        offset_dims=(), collapsed_slice_dims=(0,), start_index_map=(0,)
    )

    def bcast_last(v):
        return lax.gather(
            v,
            last15,
            dimension_numbers=bcast_dnums,
            slice_sizes=(1,),
            mode=lax.GatherScatterMode.PROMISE_IN_BOUNDS,
        )

    def src(t):
        g, h = t // 2, t % 2
        return x_hbm.at[pl.ds(base + g * _G, _G), pl.ds(h * _HC, _HC)]

    def dst(t):
        g, h = t // 2, t % 2
        return o_hbm.at[pl.ds(base + g * _G, _G), pl.ds(h * _HC, _HC)]

    in_handles = [None] * _NTASK
    out_handles = [None] * _NTASK
    in_handles[0] = pltpu.async_copy(src(0), bufs[0], in_sems[0])

    carrys = None
    for t in range(_NTASK):
        b = t % 2
        buf = bufs[b]
        if t + 1 < _NTASK:
            if t >= 1:
                out_handles[t - 1].wait()
            in_handles[t + 1] = pltpu.async_copy(
                src(t + 1), bufs[(t + 1) % 2], in_sems[(t + 1) % 2]
            )
        in_handles[t].wait()

        if t % 2 == 0:
            carrys = tuple(jnp.zeros((16,), jnp.float32) for _ in range(_G))

        def vec_body(i, cs):
            new = []
            for r in range(_G):
                v = buf[r, pl.ds(i * 16, 16)]
                out = plsc.cumsum(v) + cs[r]
                buf[r, pl.ds(i * 16, 16)] = out
                new.append(bcast_last(out))
            return tuple(new)

        carrys = plsc.parallel_loop(0, _HC // 16, unroll=2, carry=carrys)(
            vec_body
        )
        out_handles[t] = pltpu.async_copy(buf, dst(t), out_sems[b])

    out_handles[_NTASK - 2].wait()
    out_handles[_NTASK - 1].wait()


_BR = 2048
_BC = 1024
_CH = 128


def _tc_scan_kernel(x_ref, tri_ref, o_ref, carry_ref):
    j = pl.program_id(1)

    @pl.when(j == 0)
    def _():
        carry_ref[...] = jnp.zeros_like(carry_ref)

    tri = tri_ref[...]
    carry = carry_ref[:, 0:1]
    for k in range(_BC // _CH):
        xc = x_ref[:, k * _CH : (k + 1) * _CH]
        part = jax.lax.dot_general(
            xc,
            tri,
            dimension_numbers=(((1,), (0,)), ((), ())),
            precision=jax.lax.Precision.DEFAULT,
            preferred_element_type=jnp.float32,
        )
        outc = part + carry
        o_ref[:, k * _CH : (k + 1) * _CH] = outc
        carry = outc[:, _CH - 1 : _CH]
    carry_ref[...] = jnp.broadcast_to(carry, carry_ref.shape)


def _tc_cumsum(x):
    m, n = x.shape
    tri = jnp.triu(jnp.ones((_CH, _CH), dtype=jnp.float32))
    return pl.pallas_call(
        _tc_scan_kernel,
        grid=(m // _BR, n // _BC),
        in_specs=[
            pl.BlockSpec((_BR, _BC), lambda i, j: (i, j)),
            pl.BlockSpec((_CH, _CH), lambda i, j: (0, 0)),
        ],
        out_specs=pl.BlockSpec((_BR, _BC), lambda i, j: (i, j)),
        out_shape=jax.ShapeDtypeStruct((m, n), jnp.float32),
        scratch_shapes=[pltpu.VMEM((_BR, 128), jnp.float32)],
        compiler_params=pltpu.CompilerParams(
            dimension_semantics=("parallel", "arbitrary"),
        ),
    )(x, tri)


def kernel(x):
    return _sc_cumsum(x)


# speedup vs baseline: 5.2323x; 1.0233x over previous
import functools

import jax
import jax.numpy as jnp
from jax import lax
from jax.experimental import pallas as pl
from jax.experimental.pallas import tpu as pltpu
from jax.experimental.pallas import tpu_sc as plsc


_NROWS = 4096
_NCOLS = 8192
_NCORES = 2
_NSUB = 16
_NW = _NCORES * _NSUB
_RPW = _NROWS // _NW
_NV = _NCOLS // 16


_G = 8
_HC = _NCOLS // 2
_NTASK = (_RPW // _G) * 2


@functools.partial(
    pl.kernel,
    out_type=jax.ShapeDtypeStruct((_NROWS, _NCOLS), jnp.float32),
    mesh=plsc.VectorSubcoreMesh(core_axis_name="c", subcore_axis_name="s"),
    scratch_types=[
        pltpu.VMEM((_G, _HC), jnp.float32),
        pltpu.VMEM((_G, _HC), jnp.float32),
        pltpu.SemaphoreType.DMA,
        pltpu.SemaphoreType.DMA,
        pltpu.SemaphoreType.DMA,
        pltpu.SemaphoreType.DMA,
    ],
    compiler_params=pltpu.CompilerParams(needs_layout_passes=False),
)
def _sc_cumsum(x_hbm, o_hbm, buf0, buf1, in0, in1, out0, out1):
    wid = lax.axis_index("s") * _NCORES + lax.axis_index("c")
    base = wid * _RPW
    bufs = (buf0, buf1)
    in_sems = (in0, in1)
    out_sems = (out0, out1)
    last15 = jnp.full((16, 1), 15, dtype=jnp.int32)
    bcast_dnums = lax.GatherDimensionNumbers(
        offset_dims=(), collapsed_slice_dims=(0,), start_index_map=(0,)
    )

    def bcast_last(v):
        return lax.gather(
            v,
            last15,
            dimension_numbers=bcast_dnums,
            slice_sizes=(1,),
            mode=lax.GatherScatterMode.PROMISE_IN_BOUNDS,
        )

    def src(t):
        g, h = t // 2, t % 2
        return x_hbm.at[pl.ds(base + g * _G, _G), pl.ds(h * _HC, _HC)]

    def dst(t):
        g, h = t // 2, t % 2
        return o_hbm.at[pl.ds(base + g * _G, _G), pl.ds(h * _HC, _HC)]

    in_handles = [None] * _NTASK
    out_handles = [None] * _NTASK
    in_handles[0] = pltpu.async_copy(src(0), bufs[0], in_sems[0])

    carrys = None
    for t in range(_NTASK):
        b = t % 2
        buf = bufs[b]
        if t + 1 < _NTASK:
            if t >= 1:
                out_handles[t - 1].wait()
            in_handles[t + 1] = pltpu.async_copy(
                src(t + 1), bufs[(t + 1) % 2], in_sems[(t + 1) % 2]
            )
        in_handles[t].wait()

        if t % 2 == 0:
            carrys = tuple(jnp.float32(0.0) for _ in range(_G))

        s_prev = tuple(plsc.cumsum(buf[r, pl.ds(0, 16)]) for r in range(_G))

        def vec_body(i, state):
            sp, cs = state
            new_s, new_c = [], []
            for r in range(_G):
                v = buf[r, pl.ds(i * 16, 16)]
                s = plsc.cumsum(v)
                out = sp[r] + cs[r]
                buf[r, pl.ds((i - 1) * 16, 16)] = out
                new_s.append(s)
                new_c.append(out[15])
            return tuple(new_s), tuple(new_c)

        s_prev, carrys = plsc.parallel_loop(
            1, _HC // 16, unroll=2, carry=(s_prev, carrys)
        )(vec_body)

        last = _HC - 16
        new_c = []
        for r in range(_G):
            out = s_prev[r] + carrys[r]
            buf[r, pl.ds(last, 16)] = out
            new_c.append(out[15])
        carrys = tuple(new_c)
        out_handles[t] = pltpu.async_copy(buf, dst(t), out_sems[b])

    out_handles[_NTASK - 2].wait()
    out_handles[_NTASK - 1].wait()


_BR = 2048
_BC = 1024
_CH = 128


def _tc_scan_kernel(x_ref, tri_ref, o_ref, carry_ref):
    j = pl.program_id(1)

    @pl.when(j == 0)
    def _():
        carry_ref[...] = jnp.zeros_like(carry_ref)

    tri = tri_ref[...]
    carry = carry_ref[:, 0:1]
    for k in range(_BC // _CH):
        xc = x_ref[:, k * _CH : (k + 1) * _CH]
        part = jax.lax.dot_general(
            xc,
            tri,
            dimension_numbers=(((1,), (0,)), ((), ())),
            precision=jax.lax.Precision.DEFAULT,
            preferred_element_type=jnp.float32,
        )
        outc = part + carry
        o_ref[:, k * _CH : (k + 1) * _CH] = outc
        carry = outc[:, _CH - 1 : _CH]
    carry_ref[...] = jnp.broadcast_to(carry, carry_ref.shape)


def _tc_cumsum(x):
    m, n = x.shape
    tri = jnp.triu(jnp.ones((_CH, _CH), dtype=jnp.float32))
    return pl.pallas_call(
        _tc_scan_kernel,
        grid=(m // _BR, n // _BC),
        in_specs=[
            pl.BlockSpec((_BR, _BC), lambda i, j: (i, j)),
            pl.BlockSpec((_CH, _CH), lambda i, j: (0, 0)),
        ],
        out_specs=pl.BlockSpec((_BR, _BC), lambda i, j: (i, j)),
        out_shape=jax.ShapeDtypeStruct((m, n), jnp.float32),
        scratch_shapes=[pltpu.VMEM((_BR, 128), jnp.float32)],
        compiler_params=pltpu.CompilerParams(
            dimension_semantics=("parallel", "arbitrary"),
        ),
    )(x, tri)


def kernel(x):
    return _sc_cumsum(x)


# speedup vs baseline: 6.5338x; 1.2487x over previous
import functools

import jax
import jax.numpy as jnp
from jax import lax
from jax.experimental import pallas as pl
from jax.experimental.pallas import tpu as pltpu
from jax.experimental.pallas import tpu_sc as plsc


_NROWS = 4096
_NCOLS = 8192
_NCORES = 2
_NSUB = 16
_NW = _NCORES * _NSUB
_RPW = _NROWS // _NW
_NV = _NCOLS // 16


_G = 8
_HC = _NCOLS // 2
_NTASK = (_RPW // _G) * 2


@functools.partial(
    pl.kernel,
    out_type=jax.ShapeDtypeStruct((_NROWS, _NCOLS), jnp.float32),
    mesh=plsc.VectorSubcoreMesh(core_axis_name="c", subcore_axis_name="s"),
    scratch_types=[
        pltpu.VMEM((_G, _HC), jnp.float32),
        pltpu.VMEM((_G, _HC), jnp.float32),
        pltpu.VMEM((_G, _HC), jnp.float32),
        pltpu.SemaphoreType.DMA,
        pltpu.SemaphoreType.DMA,
        pltpu.SemaphoreType.DMA,
        pltpu.SemaphoreType.DMA,
        pltpu.SemaphoreType.DMA,
        pltpu.SemaphoreType.DMA,
    ],
    compiler_params=pltpu.CompilerParams(needs_layout_passes=False),
)
def _sc_cumsum(x_hbm, o_hbm, buf0, buf1, buf2, in0, in1, in2, out0, out1, out2):
    wid = lax.axis_index("s") * _NCORES + lax.axis_index("c")
    base = wid * _RPW
    bufs = (buf0, buf1, buf2)
    in_sems = (in0, in1, in2)
    out_sems = (out0, out1, out2)
    last15 = jnp.full((16, 1), 15, dtype=jnp.int32)
    bcast_dnums = lax.GatherDimensionNumbers(
        offset_dims=(), collapsed_slice_dims=(0,), start_index_map=(0,)
    )

    def bcast_last(v):
        return lax.gather(
            v,
            last15,
            dimension_numbers=bcast_dnums,
            slice_sizes=(1,),
            mode=lax.GatherScatterMode.PROMISE_IN_BOUNDS,
        )

    def src(t):
        g, h = t // 2, t % 2
        return x_hbm.at[pl.ds(base + g * _G, _G), pl.ds(h * _HC, _HC)]

    def dst(t):
        g, h = t // 2, t % 2
        return o_hbm.at[pl.ds(base + g * _G, _G), pl.ds(h * _HC, _HC)]

    in_handles = [None] * _NTASK
    out_handles = [None] * _NTASK
    in_handles[0] = pltpu.async_copy(src(0), bufs[0], in_sems[0])

    carrys = None
    for t in range(_NTASK):
        b = t % 3
        buf = bufs[b]
        if t + 1 < _NTASK:
            if t >= 2:
                out_handles[t - 2].wait()
            in_handles[t + 1] = pltpu.async_copy(
                src(t + 1), bufs[(t + 1) % 3], in_sems[(t + 1) % 3]
            )
        in_handles[t].wait()

        if t % 2 == 0:
            carrys = tuple(jnp.float32(0.0) for _ in range(_G))

        s_prev = tuple(plsc.cumsum(buf[r, pl.ds(0, 16)]) for r in range(_G))

        def vec_body(i, state):
            sp, cs = state
            new_s, new_c = [], []
            for r in range(_G):
                v = buf[r, pl.ds(i * 16, 16)]
                s = plsc.cumsum(v)
                out = sp[r] + cs[r]
                buf[r, pl.ds((i - 1) * 16, 16)] = out
                new_s.append(s)
                new_c.append(out[15])
            return tuple(new_s), tuple(new_c)

        s_prev, carrys = plsc.parallel_loop(
            1, _HC // 16, unroll=2, carry=(s_prev, carrys)
        )(vec_body)

        last = _HC - 16
        new_c = []
        for r in range(_G):
            out = s_prev[r] + carrys[r]
            buf[r, pl.ds(last, 16)] = out
            new_c.append(out[15])
        carrys = tuple(new_c)
        out_handles[t] = pltpu.async_copy(buf, dst(t), out_sems[b])

    out_handles[_NTASK - 3].wait()
    out_handles[_NTASK - 2].wait()
    out_handles[_NTASK - 1].wait()


_BR = 2048
_BC = 1024
_CH = 128


def _tc_scan_kernel(x_ref, tri_ref, o_ref, carry_ref):
    j = pl.program_id(1)

    @pl.when(j == 0)
    def _():
        carry_ref[...] = jnp.zeros_like(carry_ref)

    tri = tri_ref[...]
    carry = carry_ref[:, 0:1]
    for k in range(_BC // _CH):
        xc = x_ref[:, k * _CH : (k + 1) * _CH]
        part = jax.lax.dot_general(
            xc,
            tri,
            dimension_numbers=(((1,), (0,)), ((), ())),
            precision=jax.lax.Precision.DEFAULT,
            preferred_element_type=jnp.float32,
        )
        outc = part + carry
        o_ref[:, k * _CH : (k + 1) * _CH] = outc
        carry = outc[:, _CH - 1 : _CH]
    carry_ref[...] = jnp.broadcast_to(carry, carry_ref.shape)


def _tc_cumsum(x):
    m, n = x.shape
    tri = jnp.triu(jnp.ones((_CH, _CH), dtype=jnp.float32))
    return pl.pallas_call(
        _tc_scan_kernel,
        grid=(m // _BR, n // _BC),
        in_specs=[
            pl.BlockSpec((_BR, _BC), lambda i, j: (i, j)),
            pl.BlockSpec((_CH, _CH), lambda i, j: (0, 0)),
        ],
        out_specs=pl.BlockSpec((_BR, _BC), lambda i, j: (i, j)),
        out_shape=jax.ShapeDtypeStruct((m, n), jnp.float32),
        scratch_shapes=[pltpu.VMEM((_BR, 128), jnp.float32)],
        compiler_params=pltpu.CompilerParams(
            dimension_semantics=("parallel", "arbitrary"),
        ),
    )(x, tri)


def kernel(x):
    return _sc_cumsum(x)


# speedup vs baseline: 6.5383x; 1.0007x over previous
import functools

import jax
import jax.numpy as jnp
from jax import lax
from jax.experimental import pallas as pl
from jax.experimental.pallas import tpu as pltpu
from jax.experimental.pallas import tpu_sc as plsc

_NROWS = 4096
_NCOLS = 8192
_NCORES = 2
_NSUB = 16
_NW = _NCORES * _NSUB
_RPW = _NROWS // _NW

_G = 8
_HC = _NCOLS // 2
_NTASK = (_RPW // _G) * 2


@functools.partial(
    pl.kernel,
    out_type=jax.ShapeDtypeStruct((_NROWS, _NCOLS), jnp.float32),
    mesh=plsc.VectorSubcoreMesh(core_axis_name="c", subcore_axis_name="s"),
    scratch_types=[
        pltpu.VMEM((_G, _HC), jnp.float32),
        pltpu.VMEM((_G, _HC), jnp.float32),
        pltpu.VMEM((_G, _HC), jnp.float32),
        pltpu.SemaphoreType.DMA,
        pltpu.SemaphoreType.DMA,
        pltpu.SemaphoreType.DMA,
        pltpu.SemaphoreType.DMA,
        pltpu.SemaphoreType.DMA,
        pltpu.SemaphoreType.DMA,
    ],
    compiler_params=pltpu.CompilerParams(needs_layout_passes=False),
)
def _sc_cumsum(x_hbm, o_hbm, buf0, buf1, buf2, in0, in1, in2, out0, out1, out2):
    wid = lax.axis_index("s") * _NCORES + lax.axis_index("c")
    base = wid * _RPW
    bufs = (buf0, buf1, buf2)
    in_sems = (in0, in1, in2)
    out_sems = (out0, out1, out2)

    def src(t):
        g, h = t // 2, t % 2
        return x_hbm.at[pl.ds(base + g * _G, _G), pl.ds(h * _HC, _HC)]

    def dst(t):
        g, h = t // 2, t % 2
        return o_hbm.at[pl.ds(base + g * _G, _G), pl.ds(h * _HC, _HC)]

    in_handles = [None] * _NTASK
    out_handles = [None] * _NTASK
    in_handles[0] = pltpu.async_copy(src(0), bufs[0], in_sems[0])

    carrys = None
    for t in range(_NTASK):
        b = t % 3
        buf = bufs[b]
        if t + 1 < _NTASK:
            if t >= 2:
                out_handles[t - 2].wait()
            in_handles[t + 1] = pltpu.async_copy(
                src(t + 1), bufs[(t + 1) % 3], in_sems[(t + 1) % 3]
            )
        in_handles[t].wait()

        if t % 2 == 0:
            carrys = tuple(jnp.float32(0.0) for _ in range(_G))

        s_prev = tuple(plsc.cumsum(buf[r, pl.ds(0, 16)]) for r in range(_G))

        def vec_body(i, state):
            sp, cs = state
            new_s, new_c = [], []
            for r in range(_G):
                v = buf[r, pl.ds(i * 16, 16)]
                s = plsc.cumsum(v)
                out = sp[r] + cs[r]
                buf[r, pl.ds((i - 1) * 16, 16)] = out
                new_s.append(s)
                new_c.append(out[15])
            return tuple(new_s), tuple(new_c)

        s_prev, carrys = plsc.parallel_loop(
            1, _HC // 16, unroll=2, carry=(s_prev, carrys)
        )(vec_body)

        last = _HC - 16
        new_c = []
        for r in range(_G):
            out = s_prev[r] + carrys[r]
            buf[r, pl.ds(last, 16)] = out
            new_c.append(out[15])
        carrys = tuple(new_c)
        out_handles[t] = pltpu.async_copy(buf, dst(t), out_sems[b])

    out_handles[_NTASK - 3].wait()
    out_handles[_NTASK - 2].wait()
    out_handles[_NTASK - 1].wait()


def kernel(x):
    return _sc_cumsum(x)
